# Initial kernel scaffold; baseline (speedup 1.0000x reference)
import jax
import jax.numpy as jnp
from jax.experimental import pallas as pl


def kernel(z, edge_index_0, edge_index_1, edge_index_2, edge_index_3, edge_index_4, edge_index_5, S0, S1, S2, S3, S4, M, W1, b1, W2, b2, W3, b3, W4, b4, W5, b5, W6, b6):
    raise NotImplementedError("write your pallas kernel here")


# speedup vs baseline: 11.7146x; 11.7146x over previous
import jax
import jax.numpy as jnp
from jax import lax
from jax.experimental import pallas as pl
from jax.experimental.pallas import tpu as pltpu, tpu_sc as plsc

NT = 16
LANES = 16


def _rup(x, m):
    return (x + m - 1) // m * m


def _bcast16(idx):
    return jnp.full((LANES,), idx, dtype=jnp.int32)


def _inv_sqrt(d):
    i = plsc.bitcast(d, jnp.int32)
    u = plsc.bitcast(jnp.int32(0x5F3759DF) - (i >> 1), jnp.float32)
    for _ in range(4):
        u = u * (1.5 - 0.5 * d * u * u)
    return u


def _inv_cbrt(v):
    i = plsc.bitcast(v, jnp.int32)
    r = plsc.bitcast(jnp.int32(0x548C2B4B) - i // 3, jnp.float32)
    third = jnp.float32(1.0 / 3.0)
    for _ in range(5):
        r = r * (4.0 - v * r * r * r) * third
    return r


def _make_cheb_kernel(n, n_pad, epp, cin, cout, with_volnorm=False, m_pp=0):
    own = n_pad // NT
    ovr = own // LANES
    nc_in = cin * n_pad
    oc = cout * own
    onc = cin * own
    red_w = max(n_pad, 16)
    wb_len = _rup(6 * cin * cout + cout, 8)

    mesh = plsc.VectorSubcoreMesh(core_axis_name="c", subcore_axis_name="s")

    scratch = [
        pltpu.VMEM((nc_in,), jnp.float32),
        pltpu.VMEM((nc_in,), jnp.float32),
        pltpu.VMEM((epp,), jnp.int32),
        pltpu.VMEM((epp,), jnp.int32),
        pltpu.VMEM((epp,), jnp.float32),
        pltpu.VMEM((onc,), jnp.float32),
        pltpu.VMEM((onc,), jnp.float32),
        pltpu.VMEM((onc,), jnp.float32),
        pltpu.VMEM((max(own, 16),), jnp.float32),
        pltpu.VMEM((oc,), jnp.float32),
        pltpu.VMEM((wb_len,), jnp.float32),
        pltpu.VMEM_SHARED((NT, red_w), jnp.float32),
        pltpu.VMEM_SHARED((max(nc_in, cout * n_pad),), jnp.float32),
    ]
    if with_volnorm:
        scratch += [
            pltpu.VMEM((m_pp,), jnp.int32),
            pltpu.VMEM((m_pp,), jnp.int32),
            pltpu.VMEM((m_pp,), jnp.int32),
        ]

    def body(x_hbm, ei_hbm, wb_hbm, *rest):
        if with_volnorm:
            mt_hbm = rest[0]
            rest = rest[1:]
        out_hbm = rest[0]
        (v_full, y_full, row_b, col_b, w_b, tx_prev, tx_new, acc, tmp,
         out_own, wb_v, red, bc) = rest[1:14]
        if with_volnorm:
            ma_b, mb_b, mc_b = rest[14:17]

        cid = lax.axis_index("c")
        tid = lax.axis_index("s")
        t0 = tid * own
        ones = jnp.full((LANES,), 1.0, dtype=jnp.float32)

        def zero_ref(ref, nwords):
            z = jnp.zeros((LANES,), jnp.float32)

            def zb(i, _):
                ref[pl.ds(i * LANES, LANES)] = z
                return 0

            lax.fori_loop(0, nwords // LANES, zb, 0)

        ne = NT * epp
        pltpu.sync_copy(ei_hbm.at[pl.ds(tid * epp, epp)], row_b)
        pltpu.sync_copy(ei_hbm.at[pl.ds(ne + tid * epp, epp)], col_b)
        pltpu.sync_copy(wb_hbm, wb_v)
        if with_volnorm:
            nm = NT * m_pp
            pltpu.sync_copy(mt_hbm.at[pl.ds(tid * m_pp, m_pp)], ma_b)
            pltpu.sync_copy(mt_hbm.at[pl.ds(nm + tid * m_pp, m_pp)], mb_b)
            pltpu.sync_copy(mt_hbm.at[pl.ds(2 * nm + tid * m_pp, m_pp)], mc_b)
        zero_ref(v_full, nc_in)
        for ch in range(cin):
            pltpu.sync_copy(x_hbm.at[pl.ds(ch * n_pad, n_pad)],
                            v_full.at[pl.ds(ch * n_pad, n_pad)])

        zero_ref(y_full, n_pad)

        def deg_body(i, _):
            r = row_b[pl.ds(i * LANES, LANES)]
            plsc.addupdate_scatter(y_full, [r], ones)
            return 0

        lax.fori_loop(0, epp // LANES, deg_body, 0)

        pltpu.sync_copy(y_full.at[pl.ds(0, n_pad)],
                        red.at[tid, pl.ds(0, n_pad)])
        plsc.subcore_barrier()
        zero_ref(acc, own)

        def dred(s, _):
            pltpu.sync_copy(red.at[s, pl.ds(t0, own)], tmp.at[pl.ds(0, own)])

            def addb(j, _):
                sl = pl.ds(j * LANES, LANES)
                acc[sl] = acc[sl] + tmp[sl]
                return 0

            lax.fori_loop(0, ovr, addb, 0)
            return 0

        lax.fori_loop(0, NT, dred, 0)

        for j in range(ovr):
            sl = pl.ds(j * LANES, LANES)
            d = acc[sl]
            acc[sl] = jnp.where(d > 0.5, _inv_sqrt(jnp.maximum(d, 0.5)), 0.0)
        pltpu.sync_copy(acc.at[pl.ds(0, own)], bc.at[pl.ds(t0, own)])
        plsc.subcore_barrier()
        pltpu.sync_copy(bc.at[pl.ds(0, n_pad)], y_full.at[pl.ds(0, n_pad)])

        def wbody(i, _):
            sl = pl.ds(i * LANES, LANES)
            r = row_b[sl]
            c = col_b[sl]
            dr = plsc.load_gather(y_full, [r])
            dc = plsc.load_gather(y_full, [c])
            w_b[sl] = -(dr * dc)
            return 0

        lax.fori_loop(0, epp // LANES, wbody, 0)

        boff = 6 * cin * cout

        def bias_body(co, _):
            bv = plsc.load_gather(wb_v, [_bcast16(boff + co)])
            for j in range(ovr):
                out_own[pl.ds(co * own + j * LANES, LANES)] = bv
            return 0

        lax.fori_loop(0, cout, bias_body, 0)

        def mm_accum(k, src_own):
            def mm_body(q, _):
                ci = q // cout
                co = q - ci * cout
                wv = plsc.load_gather(wb_v, [_bcast16((k * cin + ci) * cout + co)])
                for j in range(ovr):
                    so = pl.ds(co * own + j * LANES, LANES)
                    si = pl.ds(ci * own + j * LANES, LANES)
                    out_own[so] = out_own[so] + src_own[si] * wv
                return 0

            lax.fori_loop(0, cin * cout, mm_body, 0)

        def copy_own_from_full(dst):
            for ci in range(cin):
                for j in range(ovr):
                    dst[pl.ds(ci * own + j * LANES, LANES)] = v_full[
                        pl.ds(ci * n_pad + t0 + j * LANES, LANES)]

        copy_own_from_full(tx_prev)
        mm_accum(0, tx_prev)

        for k in range(1, 6):
            zero_ref(y_full, nc_in)

            def prop_body(i, _):
                sl = pl.ds(i * LANES, LANES)
                r = row_b[sl]
                c = col_b[sl]
                wv = w_b[sl]
                for ch in range(cin):
                    off = ch * n_pad
                    vals = plsc.load_gather(v_full, [c + off]) * wv
                    plsc.addupdate_scatter(y_full, [r + off], vals)
                return 0

            lax.fori_loop(0, epp // LANES, prop_body, 0)

            zero_ref(acc, onc)
            for ci in range(cin):
                pltpu.sync_copy(y_full.at[pl.ds(ci * n_pad, n_pad)],
                                red.at[tid, pl.ds(0, n_pad)])
                plsc.subcore_barrier()

                def red_body(s, _):
                    pltpu.sync_copy(red.at[s, pl.ds(t0, own)],
                                    tmp.at[pl.ds(0, own)])

                    def addb(j, _):
                        so = pl.ds(ci * own + j * LANES, LANES)
                        acc[so] = acc[so] + tmp[pl.ds(j * LANES, LANES)]
                        return 0

                    lax.fori_loop(0, ovr, addb, 0)
                    return 0

                lax.fori_loop(0, NT, red_body, 0)
                plsc.subcore_barrier()

            if k == 1:
                for j in range(onc // LANES):
                    sl = pl.ds(j * LANES, LANES)
                    tx_new[sl] = acc[sl]
            else:
                for j in range(onc // LANES):
                    sl = pl.ds(j * LANES, LANES)
                    tx_new[sl] = 2.0 * acc[sl] - tx_prev[sl]
            copy_own_from_full(tx_prev)
            mm_accum(k, tx_new)

            for ci in range(cin):
                pltpu.sync_copy(tx_new.at[pl.ds(ci * own, own)],
                                bc.at[pl.ds(ci * n_pad + t0, own)])
            plsc.subcore_barrier()
            pltpu.sync_copy(bc.at[pl.ds(0, nc_in)], v_full.at[pl.ds(0, nc_in)])

        if not with_volnorm:
            @pl.when(cid == 0)
            def _():
                for co in range(cout):
                    pltpu.sync_copy(out_own.at[pl.ds(co * own, own)],
                                    out_hbm.at[pl.ds(co * n_pad + t0, own)])
        else:
            for co in range(cout):
                pltpu.sync_copy(out_own.at[pl.ds(co * own, own)],
                                bc.at[pl.ds(co * n_pad + t0, own)])
            plsc.subcore_barrier()
            pltpu.sync_copy(bc.at[pl.ds(0, cout * n_pad)],
                            v_full.at[pl.ds(0, cout * n_pad)])

            part = jnp.zeros((LANES,), jnp.float32)

            def tri_body(i, part):
                sl = pl.ds(i * LANES, LANES)
                ia = ma_b[sl]
                ib = mb_b[sl]
                ic = mc_b[sl]
                a0 = plsc.load_gather(v_full, [ia])
                a1 = plsc.load_gather(v_full, [ia + n_pad])
                a2 = plsc.load_gather(v_full, [ia + 2 * n_pad])
                b0 = plsc.load_gather(v_full, [ib])
                b1 = plsc.load_gather(v_full, [ib + n_pad])
                b2 = plsc.load_gather(v_full, [ib + 2 * n_pad])
                c0 = plsc.load_gather(v_full, [ic])
                c1 = plsc.load_gather(v_full, [ic + n_pad])
                c2 = plsc.load_gather(v_full, [ic + 2 * n_pad])
                det = (a0 * (b1 * c2 - b2 * c1)
                       - a1 * (b0 * c2 - b2 * c0)
                       + a2 * (b0 * c1 - b1 * c0))
                return part + jnp.abs(det)

            part = lax.fori_loop(0, m_pp // LANES, tri_body, part)
            tmp[pl.ds(0, LANES)] = part
            pltpu.sync_copy(tmp.at[pl.ds(0, LANES)], red.at[tid, pl.ds(0, LANES)])
            plsc.subcore_barrier()

            tot = jnp.zeros((LANES,), jnp.float32)

            def sum_body(s, tot):
                pltpu.sync_copy(red.at[s, pl.ds(0, LANES)], tmp.at[pl.ds(0, LANES)])
                return tot + tmp[pl.ds(0, LANES)]

            tot = lax.fori_loop(0, NT, sum_body, tot)
            vol = jnp.sum(tot, axis=0) * jnp.float32(1.0 / 6.0)
            rscale = _inv_cbrt(jnp.full((LANES,), vol, jnp.float32))

            for co in range(cout):
                for j in range(ovr):
                    sl = pl.ds(co * own + j * LANES, LANES)
                    out_own[sl] = v_full[
                        pl.ds(co * n_pad + t0 + j * LANES, LANES)] * rscale

            @pl.when(cid == 0)
            def _():
                for co in range(cout):
                    pltpu.sync_copy(out_own.at[pl.ds(co * own, own)],
                                    out_hbm.at[pl.ds(co * n_pad + t0, own)])

    return pl.kernel(
        body,
        out_type=jax.ShapeDtypeStruct((cout * n_pad,), jnp.float32),
        mesh=mesh,
        scratch_types=scratch,
        compiler_params=pltpu.CompilerParams(
            needs_layout_passes=False, use_tc_tiling_on_sc=False),
        name=f"sc_cheb_n{n}",
    )


def _tc_upsample(s_mat, t, n_pad_out, block_n):
    nc, nf = s_mat.shape
    c = t.shape[0]
    grid = (nf + block_n - 1) // block_n

    def body(t_ref, s_ref, o_ref):
        y = lax.dot_general(t_ref[...], s_ref[...],
                            (((1,), (0,)), ((), ())),
                            preferred_element_type=jnp.float32)
        o_ref[...] = jnp.where(y > 0, y, jnp.exp(y) - 1.0)

    out = pl.pallas_call(
        body,
        grid=(grid,),
        in_specs=[
            pl.BlockSpec((c, nc), lambda i: (0, 0)),
            pl.BlockSpec((nc, block_n), lambda i: (0, i)),
        ],
        out_specs=pl.BlockSpec((c, block_n), lambda i: (0, i)),
        out_shape=jax.ShapeDtypeStruct((c, nf), jnp.float32),
    )(t, s_mat)
    return jnp.pad(out, ((0, 0), (0, n_pad_out - nf)))


def _pad_edges(ei, n, epp):
    e = ei.shape[1]
    ei = ei.astype(jnp.int32)
    pad = NT * epp - e
    if pad:
        ei = jnp.concatenate(
            [ei, jnp.full((2, pad), n, dtype=jnp.int32)], axis=1)
    return ei.reshape(-1)


def _pack_wb(w, b):
    flat = jnp.concatenate([w.reshape(-1), b.reshape(-1)])
    return jnp.pad(flat, (0, _rup(flat.shape[0], 8) - flat.shape[0]))


_LEVELS = [
    (320, 5120, 1, 16, 512),
    (625, 10000, 16, 8, 768),
    (1250, 20000, 8, 4, 1280),
    (2500, 40000, 4, 2, 2560),
    (5000, 80000, 2, 3, 5120),
    (10000, 160000, 3, 3, 10240),
]

_M_PP = _rup(20000 // NT, LANES)

_CHEB = []
for _i, (_n, _e, _ci, _co, _np_) in enumerate(_LEVELS):
    _epp = _rup(_e // NT, LANES)
    _CHEB.append(_make_cheb_kernel(
        _n, _np_, _epp, _ci, _co,
        with_volnorm=(_i == 5), m_pp=_M_PP if _i == 5 else 0))


def kernel(z, edge_index_0, edge_index_1, edge_index_2, edge_index_3,
           edge_index_4, edge_index_5, S0, S1, S2, S3, S4, M,
           W1, b1, W2, b2, W3, b3, W4, b4, W5, b5, W6, b6):
    edges = [edge_index_5, edge_index_4, edge_index_3, edge_index_2,
             edge_index_1, edge_index_0]
    smats = [S4, S3, S2, S1, S0]
    ws = [(W1, b1), (W2, b2), (W3, b3), (W4, b4), (W5, b5), (W6, b6)]
    blocks = [625, 1250, 2500, 1024, 1024]

    mt = M.astype(jnp.int32).T
    mt = jnp.pad(mt, ((0, 0), (0, NT * _M_PP - mt.shape[1]))).reshape(-1)

    x = jnp.pad(z.astype(jnp.float32).T, ((0, 0), (0, 512 - 320)))
    for i, (n, e, ci, co, n_pad) in enumerate(_LEVELS):
        epp = _rup(e // NT, LANES)
        ei = _pad_edges(edges[i], n, epp)
        wb = _pack_wb(*ws[i])
        if i < 5:
            x = _CHEB[i](x.reshape(-1), ei, wb).reshape(co, n_pad)
            nxt_pad = _LEVELS[i + 1][4]
            x = _tc_upsample(smats[i], x[:, :n], nxt_pad, blocks[i])
        else:
            x = _CHEB[i](x.reshape(-1), ei, wb, mt).reshape(co, n_pad)
    return x[:, :10000].T
